# baseline (speedup 1.0000x reference)
import functools

import jax
import jax.numpy as jnp
import numpy as np
from jax import lax
from jax.experimental import pallas as pl
from jax.experimental.pallas import tpu as pltpu
from jax.experimental.pallas import tpu_sc as plsc

N = 8192
D_FEAT = 64
K = 32
H1 = 64
H2 = 128
NUM_CLOUDS = 8
L = 16
NSUB = 32
QPW = N // NSUB
QCH = 4
NCH = QPW // QCH

_R2_F = np.float32(0.2 * 0.2)
_R2_BITS = int(np.array(0.2 * 0.2, np.float32).view(np.int32))


def _sc_search_gather(posx, posy, posz, batch, bnds, G):
    mesh = plsc.VectorSubcoreMesh(core_axis_name="c", subcore_axis_name="s")

    @functools.partial(
        pl.kernel,
        mesh=mesh,
        compiler_params=pltpu.CompilerParams(needs_layout_passes=False),
        out_type=jax.ShapeDtypeStruct((N * K, 2 * D_FEAT), jnp.float32),
        scratch_types=[
            pltpu.VMEM((N + 8 * L,), jnp.float32),
            pltpu.VMEM((N + 8 * L,), jnp.float32),
            pltpu.VMEM((N + 8 * L,), jnp.float32),
            pltpu.VMEM((N + L,), jnp.int32),
            pltpu.VMEM((2 * L,), jnp.int32),
            pltpu.VMEM((N + 8 * L,), jnp.float32),
            pltpu.VMEM((N + 8 * L,), jnp.int32),
            pltpu.VMEM((QCH * K + L,), jnp.int32),
            pltpu.VMEM((QCH * K + L,), jnp.int32),
            pltpu.VMEM((QCH * K, 2 * D_FEAT), jnp.float32),
            pltpu.VMEM((QCH * K, 2 * D_FEAT), jnp.float32),
            pltpu.SemaphoreType.DMA,
            pltpu.SemaphoreType.DMA,
            pltpu.SemaphoreType.DMA,
            pltpu.SemaphoreType.DMA,
        ],
    )
    def k(posx_hbm, posy_hbm, posz_hbm, batch_hbm, bnds_hbm, g_hbm, out_hbm,
          px, py, pz, batv, bndv, cd2, cix, idxq0, idxq1, grows0, grows1,
          gs0, gs1, ws0, ws1):
        wid = lax.axis_index("s") * 2 + lax.axis_index("c")
        pltpu.sync_copy(posx_hbm, px.at[pl.ds(0, N)])
        pltpu.sync_copy(posy_hbm, py.at[pl.ds(0, N)])
        pltpu.sync_copy(posz_hbm, pz.at[pl.ds(0, N)])
        pltpu.sync_copy(batch_hbm, batv.at[pl.ds(0, N)])
        pltpu.sync_copy(bnds_hbm, bndv.at[pl.ds(0, L)])

        ii = lax.broadcasted_iota(jnp.int32, (L,), 0)
        zi = jnp.zeros((L,), jnp.int32)

        def sread(ref, idx):
            return ref[pl.ds(idx, L)][0]

        def query_body(qq, qc, idxq):
            i = wid * QPW + qc * QCH + qq
            c = sread(batv, i)
            s = sread(bndv, c)
            e = sread(bndv, c + 1)
            qx = sread(px, i)
            qy = sread(py, i)
            qz = sread(pz, i)
            ce = (s + (L - 1)) >> 4
            fl = e >> 4
            nbi = jnp.maximum(fl - ce, 0)

            def step(base, m, masked):
                lix = base + ii
                dx = px[pl.ds(base, L)] - qx
                dy = py[pl.ds(base, L)] - qy
                dz = pz[pl.ds(base, L)] - qz
                d2 = dx * dx + dy * dy + dz * dz
                val = d2 <= _R2_F
                if masked == "head":
                    val = val & (lix >= s) & (lix < ce * L) & (lix < e)
                elif masked == "tail":
                    val = val & (lix >= ce * L) & (lix < e)
                pc = plsc.all_reduce_population_count(val)[0]
                return d2, lix, val, pc

            def emit(m, vals):
                for d2, lix, val, pc in vals:
                    plsc.store_compressed(cd2.at[pl.ds(m, L)], d2, mask=val)
                    plsc.store_compressed(cix.at[pl.ds(m, L)], lix, mask=val)
                    m = m + pc
                return m

            def scan_body(blk, m):
                vals = []
                for u in range(8):
                    vals.append(step((ce + blk * 8 + u) * L, m, None))
                return emit(m, vals)

            def scan_rem(b, m):
                return emit(m, [step((ce + (nbi & ~7) + b) * L, m, None)])

            m_sc = emit(0, [step((s >> 4) * L, 0, "head")])
            m_sc = lax.fori_loop(0, nbi >> 3, scan_body, m_sc)
            m_sc = lax.fori_loop(0, nbi & 7, scan_rem, m_sc)
            m_sc = emit(m_sc, [step(fl * L, m_sc, "tail")])
            nbm = (m_sc + (L - 1)) >> 4

            def do_search():
                def count_le(tb):
                    def cb(blk, acc):
                        for u in range(2):
                            base = (blk * 2 + u) * L
                            d2b = lax.bitcast_convert_type(
                                cd2[pl.ds(base, L)], jnp.int32)
                            ok = (base + ii < m_sc) & (d2b <= tb)
                            acc = acc + plsc.all_reduce_population_count(ok)
                        return acc
                    return lax.fori_loop(0, (nbm + 1) >> 1, cb, zi)

                def bs_body(t, lohi):
                    lo, hi = lohi
                    mid = lo + ((hi - lo) >> 1)
                    ge = count_le(mid) >= K
                    return (jnp.where(ge, lo, mid + 1),
                            jnp.where(ge, mid, hi))

                lo, _ = lax.fori_loop(0, 30, bs_body, (zi, zi + _R2_BITS))
                tstar = lo

                def clt(blk, acc):
                    for u in range(2):
                        base = (blk * 2 + u) * L
                        d2b = lax.bitcast_convert_type(
                            cd2[pl.ds(base, L)], jnp.int32)
                        ok = (base + ii < m_sc) & (d2b < tstar)
                        acc = acc + plsc.all_reduce_population_count(ok)
                    return acc

                n_lt = lax.fori_loop(0, (nbm + 1) >> 1, clt, zi)
                r = K - n_lt

                def count_tie_le(icv):
                    def cb(blk, acc):
                        for u in range(2):
                            base = (blk * 2 + u) * L
                            d2b = lax.bitcast_convert_type(
                                cd2[pl.ds(base, L)], jnp.int32)
                            ixv = cix[pl.ds(base, L)]
                            ok = ((base + ii < m_sc) & (d2b == tstar)
                                  & (ixv <= icv))
                            acc = acc + plsc.all_reduce_population_count(ok)
                        return acc
                    return lax.fori_loop(0, (nbm + 1) >> 1, cb, zi)

                def bs2_body(t, lohi):
                    lo, hi = lohi
                    mid = lo + ((hi - lo) >> 1)
                    ge = count_tie_le(mid) >= r
                    return (jnp.where(ge, lo, mid + 1),
                            jnp.where(ge, mid, hi))

                icut, _ = lax.fori_loop(0, 13, bs2_body, (zi, zi + (N - 1)))
                return tstar, icut

            def do_search_fast():
                bigd = jnp.int32(0x7FFFFFFF)
                dm = [jnp.where(u * L + ii < m_sc,
                                lax.bitcast_convert_type(
                                    cd2[pl.ds(u * L, L)], jnp.int32),
                                bigd) for u in range(3)]
                xv = [cix[pl.ds(u * L, L)] for u in range(3)]

                def cle(tb):
                    acc = zi
                    for u in range(3):
                        acc = acc + plsc.all_reduce_population_count(
                            dm[u] <= tb)
                    return acc

                def bs_body(t, lohi):
                    lo, hi = lohi
                    mid = lo + ((hi - lo) >> 1)
                    ge = cle(mid) >= K
                    return (jnp.where(ge, lo, mid + 1),
                            jnp.where(ge, mid, hi))

                tstar, _ = lax.fori_loop(0, 30, bs_body, (zi, zi + _R2_BITS))

                n_lt = zi
                for u in range(3):
                    n_lt = n_lt + plsc.all_reduce_population_count(
                        dm[u] < tstar)
                r = K - n_lt

                tx = [jnp.where(dm[u] == tstar, xv[u], jnp.int32(N))
                      for u in range(3)]

                def ctle(icv):
                    acc = zi
                    for u in range(3):
                        acc = acc + plsc.all_reduce_population_count(
                            tx[u] <= icv)
                    return acc

                def bs2_body(t, lohi):
                    lo, hi = lohi
                    mid = lo + ((hi - lo) >> 1)
                    ge = ctle(mid) >= r
                    return (jnp.where(ge, lo, mid + 1),
                            jnp.where(ge, mid, hi))

                icut, _ = lax.fori_loop(0, 13, bs2_body, (zi, zi + (N - 1)))
                return tstar, icut

            def no_search():
                return zi + (_R2_BITS + 1), zi + (N - 1)

            tstar, icut = lax.cond(
                m_sc > K,
                lambda: lax.cond(m_sc <= 3 * L, do_search_fast, do_search),
                no_search)

            def selpass(b, m2):
                base = b * L
                d2b = lax.bitcast_convert_type(cd2[pl.ds(base, L)], jnp.int32)
                ixv = cix[pl.ds(base, L)]
                sel = ((base + ii < m_sc)
                       & ((d2b < tstar) | ((d2b == tstar) & (ixv <= icut))))
                pc = plsc.all_reduce_population_count(sel)[0]
                plsc.store_compressed(idxq.at[pl.ds(qq * K + m2, L)],
                                      ixv, mask=sel)
                return m2 + pc

            m2 = lax.fori_loop(0, nbm, selpass, 0)

            for kk in range(K // L):
                off = qq * K + kk * L
                cur = idxq[pl.ds(off, L)]
                idxq[pl.ds(off, L)] = jnp.where(kk * L + ii < m2, cur, zi + i)
            return 0

        def compute_chunk(qc, idxq):
            lax.fori_loop(
                0, QCH,
                lambda qq, _: query_body(qq, qc, idxq), 0)

        def out_slice(qc):
            row0 = wid * (QPW * K) + qc * (QCH * K)
            return out_hbm.at[pl.ds(row0, QCH * K)]

        bufs = ((idxq0, grows0, gs0, ws0), (idxq1, grows1, gs1, ws1))

        def group_body(gidx, _):
            for b in range(2):
                qc = gidx * 2 + b
                idxq, grows, gsem, wsem = bufs[b]
                oidxq, ogrows, ogsem, owsem = bufs[1 - b]
                idxs = idxq.at[pl.ds(0, QCH * K)]
                oidxs = oidxq.at[pl.ds(0, QCH * K)]

                @pl.when(qc >= 2)
                def _():
                    pltpu.make_async_copy(grows, out_slice(qc - 2),
                                          wsem).wait()

                compute_chunk(qc, idxq)
                pltpu.async_copy(g_hbm.at[idxs], grows, gsem)

                @pl.when(qc >= 1)
                def _():
                    pltpu.make_async_copy(g_hbm.at[oidxs], ogrows,
                                          ogsem).wait()
                    pltpu.async_copy(ogrows, out_slice(qc - 1), owsem)

            return 0

        lax.fori_loop(0, NCH // 2, group_body, 0)
        pltpu.make_async_copy(g_hbm.at[idxq1.at[pl.ds(0, QCH * K)]],
                              grows1, gs1).wait()
        pltpu.async_copy(grows1, out_slice(NCH - 1), ws1)
        pltpu.make_async_copy(grows0, out_slice(NCH - 2), ws0).wait()
        pltpu.make_async_copy(grows1, out_slice(NCH - 1), ws1).wait()

    return k(posx, posy, posz, batch, bnds, G)


def _tc_precompute(x, posP, W1a, W1bP, b1):
    BR = 512

    def body(x_ref, p_ref, wa_ref, wb_ref, b_ref, g_ref, v_ref):
        pv = jnp.dot(p_ref[...], wb_ref[...],
                     preferred_element_type=jnp.float32)
        gv = (jnp.dot(x_ref[...], wa_ref[...],
                      preferred_element_type=jnp.float32)
              + pv + b_ref[...])
        g_ref[:, :H1] = gv
        g_ref[:, H1:] = jnp.zeros_like(gv)
        v_ref[...] = pv

    return pl.pallas_call(
        body,
        grid=(N // BR,),
        in_specs=[
            pl.BlockSpec((BR, D_FEAT), lambda i: (i, 0)),
            pl.BlockSpec((BR, 8), lambda i: (i, 0)),
            pl.BlockSpec((D_FEAT, H1), lambda i: (0, 0)),
            pl.BlockSpec((8, H1), lambda i: (0, 0)),
            pl.BlockSpec((1, H1), lambda i: (0, 0)),
        ],
        out_specs=[
            pl.BlockSpec((BR, 2 * H1), lambda i: (i, 0)),
            pl.BlockSpec((BR, H1), lambda i: (i, 0)),
        ],
        out_shape=[
            jax.ShapeDtypeStruct((N, 2 * H1), jnp.float32),
            jax.ShapeDtypeStruct((N, H1), jnp.float32),
        ],
    )(x, posP, W1a, W1bP, b1)


def _tc_mlp(Gg, V, W2, b2):
    BQ = 128

    def body(gg_ref, v_ref, w2_ref, b2_ref, o_ref):
        g3 = gg_ref[:, :H1].reshape(BQ, K, H1)
        h1 = jnp.maximum(g3 - v_ref[...][:, None, :], 0.0)
        h2 = jnp.dot(h1.reshape(BQ * K, H1), w2_ref[...],
                     preferred_element_type=jnp.float32) + b2_ref[...]
        h2 = jnp.maximum(h2, 0.0)
        o_ref[...] = jnp.max(h2.reshape(BQ, K, H2), axis=1)

    return pl.pallas_call(
        body,
        grid=(N // BQ,),
        in_specs=[
            pl.BlockSpec((BQ * K, 2 * H1), lambda i: (i, 0)),
            pl.BlockSpec((BQ, H1), lambda i: (i, 0)),
            pl.BlockSpec((H1, H2), lambda i: (0, 0)),
            pl.BlockSpec((1, H2), lambda i: (0, 0)),
        ],
        out_specs=pl.BlockSpec((BQ, H2), lambda i: (i, 0)),
        out_shape=jax.ShapeDtypeStruct((N, H2), jnp.float32),
    )(Gg, V, W2, b2)


def kernel(x, pos, batch, W1, b1, W2, b2):
    W1a = W1[:D_FEAT]
    W1b = W1[D_FEAT:]
    posP = jnp.pad(pos, ((0, 0), (0, 5)))
    W1bP = jnp.pad(W1b, ((0, 5), (0, 0)))
    G, V = _tc_precompute(x, posP, W1a, W1bP, b1.reshape(1, H1))
    bnds = jnp.searchsorted(
        batch, jnp.arange(NUM_CLOUDS + 1, dtype=jnp.int32)).astype(jnp.int32)
    bnds = jnp.pad(bnds, (0, L - NUM_CLOUDS - 1))
    Gg = _sc_search_gather(pos[:, 0], pos[:, 1], pos[:, 2], batch, bnds, G)
    out = _tc_mlp(Gg, V, W2, b2.reshape(1, H2))
    return (out, pos, batch)


# speedup vs baseline: 1.0276x; 1.0276x over previous
import functools

import jax
import jax.numpy as jnp
import numpy as np
from jax import lax
from jax.experimental import pallas as pl
from jax.experimental.pallas import tpu as pltpu
from jax.experimental.pallas import tpu_sc as plsc

N = 8192
D_FEAT = 64
K = 32
H1 = 64
H2 = 128
NUM_CLOUDS = 8
L = 16
NSUB = 32
QPW = N // NSUB
QCH = 4
NCH = QPW // QCH

_R2_F = np.float32(0.2 * 0.2)
_R2_BITS = int(np.array(0.2 * 0.2, np.float32).view(np.int32))


def _sc_search_gather(posx, posy, posz, batch, bnds, G):
    mesh = plsc.VectorSubcoreMesh(core_axis_name="c", subcore_axis_name="s")

    @functools.partial(
        pl.kernel,
        mesh=mesh,
        compiler_params=pltpu.CompilerParams(needs_layout_passes=False),
        out_type=jax.ShapeDtypeStruct((N * K, 2 * D_FEAT), jnp.float32),
        scratch_types=[
            pltpu.VMEM((N + 8 * L,), jnp.float32),
            pltpu.VMEM((N + 8 * L,), jnp.float32),
            pltpu.VMEM((N + 8 * L,), jnp.float32),
            pltpu.VMEM((N + L,), jnp.int32),
            pltpu.VMEM((2 * L,), jnp.int32),
            pltpu.VMEM((N + 8 * L,), jnp.float32),
            pltpu.VMEM((N + 8 * L,), jnp.int32),
            pltpu.VMEM((QCH * K + L,), jnp.int32),
            pltpu.VMEM((QCH * K + L,), jnp.int32),
            pltpu.VMEM((QCH * K, 2 * D_FEAT), jnp.float32),
            pltpu.VMEM((QCH * K, 2 * D_FEAT), jnp.float32),
            pltpu.SemaphoreType.DMA,
            pltpu.SemaphoreType.DMA,
            pltpu.SemaphoreType.DMA,
            pltpu.SemaphoreType.DMA,
        ],
    )
    def k(posx_hbm, posy_hbm, posz_hbm, batch_hbm, bnds_hbm, g_hbm, out_hbm,
          px, py, pz, batv, bndv, cd2, cix, idxq0, idxq1, grows0, grows1,
          gs0, gs1, ws0, ws1):
        wid = lax.axis_index("s") * 2 + lax.axis_index("c")
        pltpu.sync_copy(posx_hbm, px.at[pl.ds(0, N)])
        pltpu.sync_copy(posy_hbm, py.at[pl.ds(0, N)])
        pltpu.sync_copy(posz_hbm, pz.at[pl.ds(0, N)])
        pltpu.sync_copy(batch_hbm, batv.at[pl.ds(0, N)])
        pltpu.sync_copy(bnds_hbm, bndv.at[pl.ds(0, L)])

        ii = lax.broadcasted_iota(jnp.int32, (L,), 0)
        zi = jnp.zeros((L,), jnp.int32)

        def sread(ref, idx):
            return ref[pl.ds(idx, L)][0]

        def query_body(qq, qc, idxq):
            i = wid * QPW + qc * QCH + qq
            c = sread(batv, i)
            s = sread(bndv, c)
            e = sread(bndv, c + 1)
            qx = sread(px, i)
            qy = sread(py, i)
            qz = sread(pz, i)
            ce = (s + (L - 1)) >> 4
            nbe = ((e + (L - 1)) >> 4) - ce

            def step(base, head):
                lix = base + ii
                dx = px[pl.ds(base, L)] - qx
                dy = py[pl.ds(base, L)] - qy
                dz = pz[pl.ds(base, L)] - qz
                d2 = dx * dx + dy * dy + dz * dz
                val = (d2 <= _R2_F) & (lix < e)
                if head:
                    val = val & (lix >= s) & (lix < ce * L)
                pc = plsc.all_reduce_population_count(val)[0]
                return d2, lix, val, pc

            def emit(m, vals):
                for d2, lix, val, pc in vals:
                    plsc.store_compressed(cd2.at[pl.ds(m, L)], d2, mask=val)
                    plsc.store_compressed(cix.at[pl.ds(m, L)], lix, mask=val)
                    m = m + pc
                return m

            def scan_body(blk, m):
                vals = []
                for u in range(8):
                    vals.append(step((ce + blk * 8 + u) * L, False))
                return emit(m, vals)

            m_sc = emit(0, [step((s >> 4) * L, True)])
            m_sc = lax.fori_loop(0, (nbe + 7) >> 3, scan_body, m_sc)
            nbm = (m_sc + (L - 1)) >> 4

            def do_search():
                def count_le(tb):
                    def cb(blk, acc):
                        for u in range(2):
                            base = (blk * 2 + u) * L
                            d2b = lax.bitcast_convert_type(
                                cd2[pl.ds(base, L)], jnp.int32)
                            ok = (base + ii < m_sc) & (d2b <= tb)
                            acc = acc + plsc.all_reduce_population_count(ok)
                        return acc
                    return lax.fori_loop(0, (nbm + 1) >> 1, cb, zi)

                def bs_body(t, lohi):
                    lo, hi = lohi
                    mid = lo + ((hi - lo) >> 1)
                    ge = count_le(mid) >= K
                    return (jnp.where(ge, lo, mid + 1),
                            jnp.where(ge, mid, hi))

                lo, _ = lax.fori_loop(0, 30, bs_body, (zi, zi + _R2_BITS))
                tstar = lo

                def clt(blk, acc):
                    for u in range(2):
                        base = (blk * 2 + u) * L
                        d2b = lax.bitcast_convert_type(
                            cd2[pl.ds(base, L)], jnp.int32)
                        ok = (base + ii < m_sc) & (d2b < tstar)
                        acc = acc + plsc.all_reduce_population_count(ok)
                    return acc

                n_lt = lax.fori_loop(0, (nbm + 1) >> 1, clt, zi)
                r = K - n_lt

                def count_tie_le(icv):
                    def cb(blk, acc):
                        for u in range(2):
                            base = (blk * 2 + u) * L
                            d2b = lax.bitcast_convert_type(
                                cd2[pl.ds(base, L)], jnp.int32)
                            ixv = cix[pl.ds(base, L)]
                            ok = ((base + ii < m_sc) & (d2b == tstar)
                                  & (ixv <= icv))
                            acc = acc + plsc.all_reduce_population_count(ok)
                        return acc
                    return lax.fori_loop(0, (nbm + 1) >> 1, cb, zi)

                def bs2_body(t, lohi):
                    lo, hi = lohi
                    mid = lo + ((hi - lo) >> 1)
                    ge = count_tie_le(mid) >= r
                    return (jnp.where(ge, lo, mid + 1),
                            jnp.where(ge, mid, hi))

                icut, _ = lax.fori_loop(0, 13, bs2_body, (zi, zi + (N - 1)))
                return tstar, icut

            def do_search_fast():
                bigd = jnp.int32(0x7FFFFFFF)
                dm = [jnp.where(u * L + ii < m_sc,
                                lax.bitcast_convert_type(
                                    cd2[pl.ds(u * L, L)], jnp.int32),
                                bigd) for u in range(3)]
                xv = [cix[pl.ds(u * L, L)] for u in range(3)]

                def cle(tb):
                    acc = zi
                    for u in range(3):
                        acc = acc + plsc.all_reduce_population_count(
                            dm[u] <= tb)
                    return acc

                def bs_body(t, lohi):
                    lo, hi = lohi
                    mid = lo + ((hi - lo) >> 1)
                    ge = cle(mid) >= K
                    return (jnp.where(ge, lo, mid + 1),
                            jnp.where(ge, mid, hi))

                tstar, _ = lax.fori_loop(0, 30, bs_body, (zi, zi + _R2_BITS))

                n_lt = zi
                for u in range(3):
                    n_lt = n_lt + plsc.all_reduce_population_count(
                        dm[u] < tstar)
                r = K - n_lt

                tx = [jnp.where(dm[u] == tstar, xv[u], jnp.int32(N))
                      for u in range(3)]

                def ctle(icv):
                    acc = zi
                    for u in range(3):
                        acc = acc + plsc.all_reduce_population_count(
                            tx[u] <= icv)
                    return acc

                def bs2_body(t, lohi):
                    lo, hi = lohi
                    mid = lo + ((hi - lo) >> 1)
                    ge = ctle(mid) >= r
                    return (jnp.where(ge, lo, mid + 1),
                            jnp.where(ge, mid, hi))

                icut, _ = lax.fori_loop(0, 13, bs2_body, (zi, zi + (N - 1)))
                return tstar, icut

            def no_search():
                return zi + (_R2_BITS + 1), zi + (N - 1)

            tstar, icut = lax.cond(
                m_sc > K,
                lambda: lax.cond(m_sc <= 3 * L, do_search_fast, do_search),
                no_search)

            def selpass(b, m2):
                base = b * L
                d2b = lax.bitcast_convert_type(cd2[pl.ds(base, L)], jnp.int32)
                ixv = cix[pl.ds(base, L)]
                sel = ((base + ii < m_sc)
                       & ((d2b < tstar) | ((d2b == tstar) & (ixv <= icut))))
                pc = plsc.all_reduce_population_count(sel)[0]
                plsc.store_compressed(idxq.at[pl.ds(qq * K + m2, L)],
                                      ixv, mask=sel)
                return m2 + pc

            m2 = lax.fori_loop(0, nbm, selpass, 0)

            for kk in range(K // L):
                off = qq * K + kk * L
                cur = idxq[pl.ds(off, L)]
                idxq[pl.ds(off, L)] = jnp.where(kk * L + ii < m2, cur, zi + i)
            return 0

        def compute_chunk(qc, idxq):
            lax.fori_loop(
                0, QCH,
                lambda qq, _: query_body(qq, qc, idxq), 0)

        def out_slice(qc):
            row0 = wid * (QPW * K) + qc * (QCH * K)
            return out_hbm.at[pl.ds(row0, QCH * K)]

        bufs = ((idxq0, grows0, gs0, ws0), (idxq1, grows1, gs1, ws1))

        def group_body(gidx, _):
            for b in range(2):
                qc = gidx * 2 + b
                idxq, grows, gsem, wsem = bufs[b]
                oidxq, ogrows, ogsem, owsem = bufs[1 - b]
                idxs = idxq.at[pl.ds(0, QCH * K)]
                oidxs = oidxq.at[pl.ds(0, QCH * K)]

                @pl.when(qc >= 2)
                def _():
                    pltpu.make_async_copy(grows, out_slice(qc - 2),
                                          wsem).wait()

                compute_chunk(qc, idxq)
                pltpu.async_copy(g_hbm.at[idxs], grows, gsem)

                @pl.when(qc >= 1)
                def _():
                    pltpu.make_async_copy(g_hbm.at[oidxs], ogrows,
                                          ogsem).wait()
                    pltpu.async_copy(ogrows, out_slice(qc - 1), owsem)

            return 0

        lax.fori_loop(0, NCH // 2, group_body, 0)
        pltpu.make_async_copy(g_hbm.at[idxq1.at[pl.ds(0, QCH * K)]],
                              grows1, gs1).wait()
        pltpu.async_copy(grows1, out_slice(NCH - 1), ws1)
        pltpu.make_async_copy(grows0, out_slice(NCH - 2), ws0).wait()
        pltpu.make_async_copy(grows1, out_slice(NCH - 1), ws1).wait()

    return k(posx, posy, posz, batch, bnds, G)


def _tc_precompute(x, posP, W1a, W1bP, b1):
    BR = 512

    def body(x_ref, p_ref, wa_ref, wb_ref, b_ref, g_ref, v_ref):
        pv = jnp.dot(p_ref[...], wb_ref[...],
                     preferred_element_type=jnp.float32)
        gv = (jnp.dot(x_ref[...], wa_ref[...],
                      preferred_element_type=jnp.float32)
              + pv + b_ref[...])
        g_ref[:, :H1] = gv
        g_ref[:, H1:] = jnp.zeros_like(gv)
        v_ref[...] = pv

    return pl.pallas_call(
        body,
        grid=(N // BR,),
        in_specs=[
            pl.BlockSpec((BR, D_FEAT), lambda i: (i, 0)),
            pl.BlockSpec((BR, 8), lambda i: (i, 0)),
            pl.BlockSpec((D_FEAT, H1), lambda i: (0, 0)),
            pl.BlockSpec((8, H1), lambda i: (0, 0)),
            pl.BlockSpec((1, H1), lambda i: (0, 0)),
        ],
        out_specs=[
            pl.BlockSpec((BR, 2 * H1), lambda i: (i, 0)),
            pl.BlockSpec((BR, H1), lambda i: (i, 0)),
        ],
        out_shape=[
            jax.ShapeDtypeStruct((N, 2 * H1), jnp.float32),
            jax.ShapeDtypeStruct((N, H1), jnp.float32),
        ],
    )(x, posP, W1a, W1bP, b1)


def _tc_mlp(Gg, V, W2, b2):
    BQ = 128

    def body(gg_ref, v_ref, w2_ref, b2_ref, o_ref):
        g3 = gg_ref[:, :H1].reshape(BQ, K, H1)
        h1 = jnp.maximum(g3 - v_ref[...][:, None, :], 0.0)
        h2 = jnp.dot(h1.reshape(BQ * K, H1), w2_ref[...],
                     preferred_element_type=jnp.float32) + b2_ref[...]
        h2 = jnp.maximum(h2, 0.0)
        o_ref[...] = jnp.max(h2.reshape(BQ, K, H2), axis=1)

    return pl.pallas_call(
        body,
        grid=(N // BQ,),
        in_specs=[
            pl.BlockSpec((BQ * K, 2 * H1), lambda i: (i, 0)),
            pl.BlockSpec((BQ, H1), lambda i: (i, 0)),
            pl.BlockSpec((H1, H2), lambda i: (0, 0)),
            pl.BlockSpec((1, H2), lambda i: (0, 0)),
        ],
        out_specs=pl.BlockSpec((BQ, H2), lambda i: (i, 0)),
        out_shape=jax.ShapeDtypeStruct((N, H2), jnp.float32),
    )(Gg, V, W2, b2)


def kernel(x, pos, batch, W1, b1, W2, b2):
    W1a = W1[:D_FEAT]
    W1b = W1[D_FEAT:]
    posP = jnp.pad(pos, ((0, 0), (0, 5)))
    W1bP = jnp.pad(W1b, ((0, 5), (0, 0)))
    G, V = _tc_precompute(x, posP, W1a, W1bP, b1.reshape(1, H1))
    bnds = jnp.searchsorted(
        batch, jnp.arange(NUM_CLOUDS + 1, dtype=jnp.int32)).astype(jnp.int32)
    bnds = jnp.pad(bnds, (0, L - NUM_CLOUDS - 1))
    Gg = _sc_search_gather(pos[:, 0], pos[:, 1], pos[:, 2], batch, bnds, G)
    out = _tc_mlp(Gg, V, W2, b2.reshape(1, H2))
    return (out, pos, batch)


# speedup vs baseline: 1.0873x; 1.0580x over previous
import functools

import jax
import jax.numpy as jnp
import numpy as np
from jax import lax
from jax.experimental import pallas as pl
from jax.experimental.pallas import tpu as pltpu
from jax.experimental.pallas import tpu_sc as plsc

N = 8192
D_FEAT = 64
K = 32
H1 = 64
H2 = 128
NUM_CLOUDS = 8
L = 16
NSUB = 32
QPW = N // NSUB
QCH = 4
NCH = QPW // QCH

_R2_F = np.float32(0.2 * 0.2)
_R2_BITS = int(np.array(0.2 * 0.2, np.float32).view(np.int32))


def _sc_search_gather(posx, posy, posz, batch, bnds, G):
    mesh = plsc.VectorSubcoreMesh(core_axis_name="c", subcore_axis_name="s")

    @functools.partial(
        pl.kernel,
        mesh=mesh,
        compiler_params=pltpu.CompilerParams(needs_layout_passes=False),
        out_type=jax.ShapeDtypeStruct((N * K, 2 * D_FEAT), jnp.float32),
        scratch_types=[
            pltpu.VMEM((N + 8 * L,), jnp.float32),
            pltpu.VMEM((N + 8 * L,), jnp.float32),
            pltpu.VMEM((N + 8 * L,), jnp.float32),
            pltpu.VMEM((N + L,), jnp.int32),
            pltpu.VMEM((2 * L,), jnp.int32),
            pltpu.VMEM((N + 8 * L,), jnp.float32),
            pltpu.VMEM((N + 8 * L,), jnp.int32),
            pltpu.VMEM((QCH * K + L,), jnp.int32),
            pltpu.VMEM((QCH * K + L,), jnp.int32),
            pltpu.VMEM((QCH * K, 2 * D_FEAT), jnp.float32),
            pltpu.VMEM((QCH * K, 2 * D_FEAT), jnp.float32),
            pltpu.SemaphoreType.DMA,
            pltpu.SemaphoreType.DMA,
            pltpu.SemaphoreType.DMA,
            pltpu.SemaphoreType.DMA,
        ],
    )
    def k(posx_hbm, posy_hbm, posz_hbm, batch_hbm, bnds_hbm, g_hbm, out_hbm,
          px, py, pz, batv, bndv, cd2, cix, idxq0, idxq1, grows0, grows1,
          gs0, gs1, ws0, ws1):
        wid = lax.axis_index("s") * 2 + lax.axis_index("c")
        pltpu.sync_copy(posx_hbm, px.at[pl.ds(0, N)])
        pltpu.sync_copy(posy_hbm, py.at[pl.ds(0, N)])
        pltpu.sync_copy(posz_hbm, pz.at[pl.ds(0, N)])
        pltpu.sync_copy(batch_hbm, batv.at[pl.ds(0, N)])
        pltpu.sync_copy(bnds_hbm, bndv.at[pl.ds(0, L)])

        ii = lax.broadcasted_iota(jnp.int32, (L,), 0)
        zi = jnp.zeros((L,), jnp.int32)

        def sread(ref, idx):
            return ref[pl.ds(idx, L)][0]

        def query_body(qq, qc, idxq):
            i = wid * QPW + qc * QCH + qq
            c = sread(batv, i)
            s = sread(bndv, c)
            e = sread(bndv, c + 1)
            qx = sread(px, i)
            qy = sread(py, i)
            qz = sread(pz, i)
            ce = (s + (L - 1)) >> 4
            nbe = ((e + (L - 1)) >> 4) - ce

            def step(base, head):
                lix = base + ii
                dx = px[pl.ds(base, L)] - qx
                dy = py[pl.ds(base, L)] - qy
                dz = pz[pl.ds(base, L)] - qz
                d2 = dx * dx + dy * dy + dz * dz
                val = (d2 <= _R2_F) & (lix < e)
                if head:
                    val = val & (lix >= s) & (lix < ce * L)
                pc = plsc.all_reduce_population_count(val)[0]
                return d2, lix, val, pc

            def emit(m, vals):
                for d2, lix, val, pc in vals:
                    plsc.store_compressed(cd2.at[pl.ds(m, L)], d2, mask=val)
                    plsc.store_compressed(cix.at[pl.ds(m, L)], lix, mask=val)
                    m = m + pc
                return m

            def scan_body(blk, m):
                vals = []
                for u in range(8):
                    vals.append(step((ce + blk * 8 + u) * L, False))
                return emit(m, vals)

            m_sc = emit(0, [step((s >> 4) * L, True)])
            m_sc = lax.fori_loop(0, (nbe + 7) >> 3, scan_body, m_sc)
            nbm = (m_sc + (L - 1)) >> 4

            def do_search():
                def count_le(tb):
                    def cb(blk, acc):
                        for u in range(2):
                            base = (blk * 2 + u) * L
                            d2b = lax.bitcast_convert_type(
                                cd2[pl.ds(base, L)], jnp.int32)
                            ok = (base + ii < m_sc) & (d2b <= tb)
                            acc = acc + plsc.all_reduce_population_count(ok)
                        return acc
                    return lax.fori_loop(0, (nbm + 1) >> 1, cb, zi)

                def bs_body(t, lohi):
                    lo, hi = lohi
                    mid = lo + ((hi - lo) >> 1)
                    ge = count_le(mid) >= K
                    return (jnp.where(ge, lo, mid + 1),
                            jnp.where(ge, mid, hi))

                lo, _ = lax.fori_loop(0, 30, bs_body, (zi, zi + _R2_BITS))
                tstar = lo

                def clt(blk, acc):
                    for u in range(2):
                        base = (blk * 2 + u) * L
                        d2b = lax.bitcast_convert_type(
                            cd2[pl.ds(base, L)], jnp.int32)
                        ok = (base + ii < m_sc) & (d2b < tstar)
                        acc = acc + plsc.all_reduce_population_count(ok)
                    return acc

                n_lt = lax.fori_loop(0, (nbm + 1) >> 1, clt, zi)
                r = K - n_lt

                def count_tie_le(icv):
                    def cb(blk, acc):
                        for u in range(2):
                            base = (blk * 2 + u) * L
                            d2b = lax.bitcast_convert_type(
                                cd2[pl.ds(base, L)], jnp.int32)
                            ixv = cix[pl.ds(base, L)]
                            ok = ((base + ii < m_sc) & (d2b == tstar)
                                  & (ixv <= icv))
                            acc = acc + plsc.all_reduce_population_count(ok)
                        return acc
                    return lax.fori_loop(0, (nbm + 1) >> 1, cb, zi)

                def bs2_body(t, lohi):
                    lo, hi = lohi
                    mid = lo + ((hi - lo) >> 1)
                    ge = count_tie_le(mid) >= r
                    return (jnp.where(ge, lo, mid + 1),
                            jnp.where(ge, mid, hi))

                icut, _ = lax.fori_loop(0, 13, bs2_body, (zi, zi + (N - 1)))
                return tstar, icut

            def do_search_fast():
                bigd = jnp.int32(0x7FFFFFFF)
                dm = [jnp.where(u * L + ii < m_sc,
                                lax.bitcast_convert_type(
                                    cd2[pl.ds(u * L, L)], jnp.int32),
                                bigd) for u in range(3)]
                xv = [cix[pl.ds(u * L, L)] for u in range(3)]

                def cle(tb):
                    acc = zi
                    for u in range(3):
                        acc = acc + plsc.all_reduce_population_count(
                            dm[u] <= tb)
                    return acc

                def bs_body(t, lohi):
                    lo, hi = lohi
                    mid = lo + ((hi - lo) >> 1)
                    ge = cle(mid) >= K
                    return (jnp.where(ge, lo, mid + 1),
                            jnp.where(ge, mid, hi))

                tstar, _ = lax.fori_loop(0, 30, bs_body, (zi, zi + _R2_BITS))

                n_lt = zi
                for u in range(3):
                    n_lt = n_lt + plsc.all_reduce_population_count(
                        dm[u] < tstar)
                r = K - n_lt

                tx = [jnp.where(dm[u] == tstar, xv[u], jnp.int32(N))
                      for u in range(3)]

                def ctle(icv):
                    acc = zi
                    for u in range(3):
                        acc = acc + plsc.all_reduce_population_count(
                            tx[u] <= icv)
                    return acc

                def bs2_body(t, lohi):
                    lo, hi = lohi
                    mid = lo + ((hi - lo) >> 1)
                    ge = ctle(mid) >= r
                    return (jnp.where(ge, lo, mid + 1),
                            jnp.where(ge, mid, hi))

                icut, _ = lax.fori_loop(0, 13, bs2_body, (zi, zi + (N - 1)))
                return tstar, icut

            def no_search():
                return zi + (_R2_BITS + 1), zi + (N - 1)

            tstar, icut = lax.cond(
                m_sc > K,
                lambda: lax.cond(m_sc <= 3 * L, do_search_fast, do_search),
                no_search)

            def selpass(b, m2):
                base = b * L
                d2b = lax.bitcast_convert_type(cd2[pl.ds(base, L)], jnp.int32)
                ixv = cix[pl.ds(base, L)]
                sel = ((base + ii < m_sc)
                       & ((d2b < tstar) | ((d2b == tstar) & (ixv <= icut))))
                pc = plsc.all_reduce_population_count(sel)[0]
                plsc.store_compressed(idxq.at[pl.ds(qq * K + m2, L)],
                                      ixv, mask=sel)
                return m2 + pc

            m2 = lax.fori_loop(0, nbm, selpass, 0)

            for kk in range(K // L):
                off = qq * K + kk * L
                cur = idxq[pl.ds(off, L)]
                idxq[pl.ds(off, L)] = jnp.where(kk * L + ii < m2, cur, zi + i)
            return 0

        def compute_chunk(qc, idxq):
            lax.fori_loop(
                0, QCH,
                lambda qq, _: query_body(qq, qc, idxq), 0)

        def out_slice(qc):
            row0 = wid * (QPW * K) + qc * (QCH * K)
            return out_hbm.at[pl.ds(row0, QCH * K)]

        bufs = ((idxq0, grows0, gs0, ws0), (idxq1, grows1, gs1, ws1))

        def group_body(gidx, _):
            for b in range(2):
                qc = gidx * 2 + b
                idxq, grows, gsem, wsem = bufs[b]
                oidxq, ogrows, ogsem, owsem = bufs[1 - b]
                idxs = idxq.at[pl.ds(0, QCH * K)]
                oidxs = oidxq.at[pl.ds(0, QCH * K)]

                @pl.when(qc >= 2)
                def _():
                    pltpu.make_async_copy(grows, out_slice(qc - 2),
                                          wsem).wait()

                compute_chunk(qc, idxq)
                pltpu.async_copy(g_hbm.at[idxs], grows, gsem)

                @pl.when(qc >= 1)
                def _():
                    pltpu.make_async_copy(g_hbm.at[oidxs], ogrows,
                                          ogsem).wait()
                    pltpu.async_copy(ogrows, out_slice(qc - 1), owsem)

            return 0

        lax.fori_loop(0, NCH // 2, group_body, 0)
        pltpu.make_async_copy(g_hbm.at[idxq1.at[pl.ds(0, QCH * K)]],
                              grows1, gs1).wait()
        pltpu.async_copy(grows1, out_slice(NCH - 1), ws1)
        pltpu.make_async_copy(grows0, out_slice(NCH - 2), ws0).wait()
        pltpu.make_async_copy(grows1, out_slice(NCH - 1), ws1).wait()

    return k(posx, posy, posz, batch, bnds, G)


def _tc_precompute(x, posP, W1a, W1bP, b1):
    BR = 512

    def body(x_ref, p_ref, wa_ref, wb_ref, b_ref, g_ref, v_ref):
        pv = jnp.dot(p_ref[...], wb_ref[...],
                     preferred_element_type=jnp.float32)
        gv = (jnp.dot(x_ref[...], wa_ref[...],
                      preferred_element_type=jnp.float32)
              + pv + b_ref[...])
        g_ref[:, :H1] = gv
        g_ref[:, H1:] = jnp.zeros_like(gv)
        v_ref[...] = pv

    return pl.pallas_call(
        body,
        grid=(N // BR,),
        in_specs=[
            pl.BlockSpec((BR, D_FEAT), lambda i: (i, 0)),
            pl.BlockSpec((BR, 8), lambda i: (i, 0)),
            pl.BlockSpec((D_FEAT, H1), lambda i: (0, 0)),
            pl.BlockSpec((8, H1), lambda i: (0, 0)),
            pl.BlockSpec((1, H1), lambda i: (0, 0)),
        ],
        out_specs=[
            pl.BlockSpec((BR, 2 * H1), lambda i: (i, 0)),
            pl.BlockSpec((BR, H1), lambda i: (i, 0)),
        ],
        out_shape=[
            jax.ShapeDtypeStruct((N, 2 * H1), jnp.float32),
            jax.ShapeDtypeStruct((N, H1), jnp.float32),
        ],
    )(x, posP, W1a, W1bP, b1)


def _tc_mlp(Gg, V, W2, b2):
    BQ = 256

    def body(gg_ref, v_ref, w2_ref, b2_ref, o_ref):
        g3 = gg_ref[:, :H1].reshape(BQ, K, H1)
        h1 = jnp.maximum(g3 - v_ref[...][:, None, :], 0.0)
        h2 = jnp.dot(h1.reshape(BQ * K, H1), w2_ref[...],
                     preferred_element_type=jnp.float32) + b2_ref[...]
        h2 = jnp.maximum(h2, 0.0)
        o_ref[...] = jnp.max(h2.reshape(BQ, K, H2), axis=1)

    return pl.pallas_call(
        body,
        grid=(N // BQ,),
        in_specs=[
            pl.BlockSpec((BQ * K, 2 * H1), lambda i: (i, 0)),
            pl.BlockSpec((BQ, H1), lambda i: (i, 0)),
            pl.BlockSpec((H1, H2), lambda i: (0, 0)),
            pl.BlockSpec((1, H2), lambda i: (0, 0)),
        ],
        out_specs=pl.BlockSpec((BQ, H2), lambda i: (i, 0)),
        out_shape=jax.ShapeDtypeStruct((N, H2), jnp.float32),
    )(Gg, V, W2, b2)


def kernel(x, pos, batch, W1, b1, W2, b2):
    W1a = W1[:D_FEAT]
    W1b = W1[D_FEAT:]
    posP = jnp.pad(pos, ((0, 0), (0, 5)))
    W1bP = jnp.pad(W1b, ((0, 5), (0, 0)))
    G, V = _tc_precompute(x, posP, W1a, W1bP, b1.reshape(1, H1))
    bnds = jnp.searchsorted(
        batch, jnp.arange(NUM_CLOUDS + 1, dtype=jnp.int32)).astype(jnp.int32)
    bnds = jnp.pad(bnds, (0, L - NUM_CLOUDS - 1))
    Gg = _sc_search_gather(pos[:, 0], pos[:, 1], pos[:, 2], batch, bnds, G)
    out = _tc_mlp(Gg, V, W2, b2.reshape(1, H2))
    return (out, pos, batch)


# speedup vs baseline: 1.1174x; 1.0277x over previous
import functools

import jax
import jax.numpy as jnp
import numpy as np
from jax import lax
from jax.experimental import pallas as pl
from jax.experimental.pallas import tpu as pltpu
from jax.experimental.pallas import tpu_sc as plsc

N = 8192
D_FEAT = 64
K = 32
H1 = 64
H2 = 128
NUM_CLOUDS = 8
L = 16
NSUB = 32
QPW = N // NSUB
QCH = 4
NCH = QPW // QCH

_R2_F = np.float32(0.2 * 0.2)
_R2_BITS = int(np.array(0.2 * 0.2, np.float32).view(np.int32))


def _sc_search_gather(posx, posy, posz, batch, bnds, G):
    mesh = plsc.VectorSubcoreMesh(core_axis_name="c", subcore_axis_name="s")

    @functools.partial(
        pl.kernel,
        mesh=mesh,
        compiler_params=pltpu.CompilerParams(needs_layout_passes=False),
        out_type=jax.ShapeDtypeStruct((N * K, 2 * D_FEAT), jnp.float32),
        scratch_types=[
            pltpu.VMEM((N + 8 * L,), jnp.float32),
            pltpu.VMEM((N + 8 * L,), jnp.float32),
            pltpu.VMEM((N + 8 * L,), jnp.float32),
            pltpu.VMEM((N + L,), jnp.int32),
            pltpu.VMEM((2 * L,), jnp.int32),
            pltpu.VMEM((N + 8 * L,), jnp.float32),
            pltpu.VMEM((N + 8 * L,), jnp.int32),
            pltpu.VMEM((QCH * K + L,), jnp.int32),
            pltpu.VMEM((QCH * K + L,), jnp.int32),
            pltpu.VMEM((QCH * K, 2 * D_FEAT), jnp.float32),
            pltpu.VMEM((QCH * K, 2 * D_FEAT), jnp.float32),
            pltpu.SemaphoreType.DMA,
            pltpu.SemaphoreType.DMA,
            pltpu.SemaphoreType.DMA,
            pltpu.SemaphoreType.DMA,
        ],
    )
    def k(posx_hbm, posy_hbm, posz_hbm, batch_hbm, bnds_hbm, g_hbm, out_hbm,
          px, py, pz, batv, bndv, cd2, cix, idxq0, idxq1, grows0, grows1,
          gs0, gs1, ws0, ws1):
        wid = lax.axis_index("s") * 2 + lax.axis_index("c")
        pltpu.sync_copy(posx_hbm, px.at[pl.ds(0, N)])
        pltpu.sync_copy(posy_hbm, py.at[pl.ds(0, N)])
        pltpu.sync_copy(posz_hbm, pz.at[pl.ds(0, N)])
        pltpu.sync_copy(batch_hbm, batv.at[pl.ds(0, N)])
        pltpu.sync_copy(bnds_hbm, bndv.at[pl.ds(0, L)])

        ii = lax.broadcasted_iota(jnp.int32, (L,), 0)
        zi = jnp.zeros((L,), jnp.int32)

        def sread(ref, idx):
            return ref[pl.ds(idx, L)][0]

        def query_body(qq, qc, idxq):
            i = wid * QPW + qc * QCH + qq
            c = sread(batv, i)
            s = sread(bndv, c)
            e = sread(bndv, c + 1)
            qx = sread(px, i)
            qy = sread(py, i)
            qz = sread(pz, i)
            ce = (s + (L - 1)) >> 4
            nbe = ((e + (L - 1)) >> 4) - ce

            def step(base, head):
                lix = base + ii
                dx = px[pl.ds(base, L)] - qx
                dy = py[pl.ds(base, L)] - qy
                dz = pz[pl.ds(base, L)] - qz
                d2 = dx * dx + dy * dy + dz * dz
                val = (d2 <= _R2_F) & (lix < e)
                if head:
                    val = val & (lix >= s) & (lix < ce * L)
                pc = plsc.all_reduce_population_count(val)[0]
                return d2, lix, val, pc

            def emit(m, vals):
                for d2, lix, val, pc in vals:
                    plsc.store_compressed(cd2.at[pl.ds(m, L)], d2, mask=val)
                    plsc.store_compressed(cix.at[pl.ds(m, L)], lix, mask=val)
                    m = m + pc
                return m

            def scan_body(blk, m):
                vals = []
                for u in range(8):
                    vals.append(step((ce + blk * 8 + u) * L, False))
                return emit(m, vals)

            m_sc = emit(0, [step((s >> 4) * L, True)])
            m_sc = lax.fori_loop(0, (nbe + 7) >> 3, scan_body, m_sc)
            nbm = (m_sc + (L - 1)) >> 4

            def do_search():
                def count_le(tb):
                    def cb(blk, acc):
                        for u in range(2):
                            base = (blk * 2 + u) * L
                            d2b = lax.bitcast_convert_type(
                                cd2[pl.ds(base, L)], jnp.int32)
                            ok = (base + ii < m_sc) & (d2b <= tb)
                            acc = acc + plsc.all_reduce_population_count(ok)
                        return acc
                    return lax.fori_loop(0, (nbm + 1) >> 1, cb, zi)

                def bs_body(t, lohi):
                    lo, hi = lohi
                    mid = lo + ((hi - lo) >> 1)
                    ge = count_le(mid) >= K
                    return (jnp.where(ge, lo, mid + 1),
                            jnp.where(ge, mid, hi))

                lo, _ = lax.fori_loop(0, 30, bs_body, (zi, zi + _R2_BITS))
                tstar = lo

                def clt(blk, acc):
                    for u in range(2):
                        base = (blk * 2 + u) * L
                        d2b = lax.bitcast_convert_type(
                            cd2[pl.ds(base, L)], jnp.int32)
                        ok = (base + ii < m_sc) & (d2b < tstar)
                        acc = acc + plsc.all_reduce_population_count(ok)
                    return acc

                n_lt = lax.fori_loop(0, (nbm + 1) >> 1, clt, zi)
                r = K - n_lt

                def count_tie_le(icv):
                    def cb(blk, acc):
                        for u in range(2):
                            base = (blk * 2 + u) * L
                            d2b = lax.bitcast_convert_type(
                                cd2[pl.ds(base, L)], jnp.int32)
                            ixv = cix[pl.ds(base, L)]
                            ok = ((base + ii < m_sc) & (d2b == tstar)
                                  & (ixv <= icv))
                            acc = acc + plsc.all_reduce_population_count(ok)
                        return acc
                    return lax.fori_loop(0, (nbm + 1) >> 1, cb, zi)

                def bs2_body(t, lohi):
                    lo, hi = lohi
                    mid = lo + ((hi - lo) >> 1)
                    ge = count_tie_le(mid) >= r
                    return (jnp.where(ge, lo, mid + 1),
                            jnp.where(ge, mid, hi))

                icut, _ = lax.fori_loop(0, 13, bs2_body, (zi, zi + (N - 1)))
                return tstar, icut

            def do_search_fast():
                bigd = jnp.int32(0x7FFFFFFF)
                dm = [jnp.where(u * L + ii < m_sc,
                                lax.bitcast_convert_type(
                                    cd2[pl.ds(u * L, L)], jnp.int32),
                                bigd) for u in range(3)]
                xv = [cix[pl.ds(u * L, L)] for u in range(3)]

                def cle(tb):
                    acc = zi
                    for u in range(3):
                        acc = acc + plsc.all_reduce_population_count(
                            dm[u] <= tb)
                    return acc

                def bs_body(t, lohi):
                    lo, hi = lohi
                    mid = lo + ((hi - lo) >> 1)
                    ge = cle(mid) >= K
                    return (jnp.where(ge, lo, mid + 1),
                            jnp.where(ge, mid, hi))

                tstar, _ = lax.fori_loop(0, 30, bs_body, (zi, zi + _R2_BITS))

                n_lt = zi
                for u in range(3):
                    n_lt = n_lt + plsc.all_reduce_population_count(
                        dm[u] < tstar)
                r = K - n_lt

                tx = [jnp.where(dm[u] == tstar, xv[u], jnp.int32(N))
                      for u in range(3)]

                def ctle(icv):
                    acc = zi
                    for u in range(3):
                        acc = acc + plsc.all_reduce_population_count(
                            tx[u] <= icv)
                    return acc

                def bs2_body(t, lohi):
                    lo, hi = lohi
                    mid = lo + ((hi - lo) >> 1)
                    ge = ctle(mid) >= r
                    return (jnp.where(ge, lo, mid + 1),
                            jnp.where(ge, mid, hi))

                icut, _ = lax.fori_loop(0, 13, bs2_body, (zi, zi + (N - 1)))
                return tstar, icut

            def no_search():
                return zi + (_R2_BITS + 1), zi + (N - 1)

            tstar, icut = lax.cond(
                m_sc > K,
                lambda: lax.cond(m_sc <= 3 * L, do_search_fast, do_search),
                no_search)

            def selpass(b, m2):
                base = b * L
                d2b = lax.bitcast_convert_type(cd2[pl.ds(base, L)], jnp.int32)
                ixv = cix[pl.ds(base, L)]
                sel = ((base + ii < m_sc)
                       & ((d2b < tstar) | ((d2b == tstar) & (ixv <= icut))))
                pc = plsc.all_reduce_population_count(sel)[0]
                plsc.store_compressed(idxq.at[pl.ds(qq * K + m2, L)],
                                      ixv, mask=sel)
                return m2 + pc

            m2 = lax.fori_loop(0, nbm, selpass, 0)

            for kk in range(K // L):
                off = qq * K + kk * L
                cur = idxq[pl.ds(off, L)]
                idxq[pl.ds(off, L)] = jnp.where(kk * L + ii < m2, cur, zi + i)
            return 0

        def compute_chunk(qc, idxq):
            lax.fori_loop(
                0, QCH,
                lambda qq, _: query_body(qq, qc, idxq), 0)

        def out_slice(qc):
            row0 = wid * (QPW * K) + qc * (QCH * K)
            return out_hbm.at[pl.ds(row0, QCH * K)]

        bufs = ((idxq0, grows0, gs0, ws0), (idxq1, grows1, gs1, ws1))

        def group_body(gidx, _):
            for b in range(2):
                qc = gidx * 2 + b
                idxq, grows, gsem, wsem = bufs[b]
                oidxq, ogrows, ogsem, owsem = bufs[1 - b]
                idxs = idxq.at[pl.ds(0, QCH * K)]
                oidxs = oidxq.at[pl.ds(0, QCH * K)]

                @pl.when(qc >= 2)
                def _():
                    pltpu.make_async_copy(grows, out_slice(qc - 2),
                                          wsem).wait()

                compute_chunk(qc, idxq)
                pltpu.async_copy(g_hbm.at[idxs], grows, gsem)

                @pl.when(qc >= 1)
                def _():
                    pltpu.make_async_copy(g_hbm.at[oidxs], ogrows,
                                          ogsem).wait()
                    pltpu.async_copy(ogrows, out_slice(qc - 1), owsem)

            return 0

        lax.fori_loop(0, NCH // 2, group_body, 0)
        pltpu.make_async_copy(g_hbm.at[idxq1.at[pl.ds(0, QCH * K)]],
                              grows1, gs1).wait()
        pltpu.async_copy(grows1, out_slice(NCH - 1), ws1)
        pltpu.make_async_copy(grows0, out_slice(NCH - 2), ws0).wait()
        pltpu.make_async_copy(grows1, out_slice(NCH - 1), ws1).wait()

    return k(posx, posy, posz, batch, bnds, G)


def _tc_precompute(x, posP, W1a, W1bP, b1):
    BR = 512

    def body(x_ref, p_ref, wa_ref, wb_ref, b_ref, g_ref, v_ref):
        pv = jnp.dot(p_ref[...], wb_ref[...],
                     preferred_element_type=jnp.float32)
        gv = (jnp.dot(x_ref[...], wa_ref[...],
                      preferred_element_type=jnp.float32)
              + pv + b_ref[...])
        g_ref[:, :H1] = gv
        g_ref[:, H1:] = jnp.zeros_like(gv)
        v_ref[...] = pv

    return pl.pallas_call(
        body,
        grid=(N // BR,),
        in_specs=[
            pl.BlockSpec((BR, D_FEAT), lambda i: (i, 0)),
            pl.BlockSpec((BR, 8), lambda i: (i, 0)),
            pl.BlockSpec((D_FEAT, H1), lambda i: (0, 0)),
            pl.BlockSpec((8, H1), lambda i: (0, 0)),
            pl.BlockSpec((1, H1), lambda i: (0, 0)),
        ],
        out_specs=[
            pl.BlockSpec((BR, 2 * H1), lambda i: (i, 0)),
            pl.BlockSpec((BR, H1), lambda i: (i, 0)),
        ],
        out_shape=[
            jax.ShapeDtypeStruct((N, 2 * H1), jnp.float32),
            jax.ShapeDtypeStruct((N, H1), jnp.float32),
        ],
    )(x, posP, W1a, W1bP, b1)


def _tc_mlp(Gg, V, W2, b2):
    BQ = 512

    def body(gg_ref, v_ref, w2_ref, b2_ref, o_ref):
        g3 = gg_ref[:, :H1].reshape(BQ, K, H1)
        h1 = jnp.maximum(g3 - v_ref[...][:, None, :], 0.0)
        h2 = jnp.dot(h1.reshape(BQ * K, H1), w2_ref[...],
                     preferred_element_type=jnp.float32) + b2_ref[...]
        h2 = jnp.maximum(h2, 0.0)
        o_ref[...] = jnp.max(h2.reshape(BQ, K, H2), axis=1)

    return pl.pallas_call(
        body,
        grid=(N // BQ,),
        in_specs=[
            pl.BlockSpec((BQ * K, 2 * H1), lambda i: (i, 0)),
            pl.BlockSpec((BQ, H1), lambda i: (i, 0)),
            pl.BlockSpec((H1, H2), lambda i: (0, 0)),
            pl.BlockSpec((1, H2), lambda i: (0, 0)),
        ],
        out_specs=pl.BlockSpec((BQ, H2), lambda i: (i, 0)),
        out_shape=jax.ShapeDtypeStruct((N, H2), jnp.float32),
    )(Gg, V, W2, b2)


def kernel(x, pos, batch, W1, b1, W2, b2):
    W1a = W1[:D_FEAT]
    W1b = W1[D_FEAT:]
    posP = jnp.pad(pos, ((0, 0), (0, 5)))
    W1bP = jnp.pad(W1b, ((0, 5), (0, 0)))
    G, V = _tc_precompute(x, posP, W1a, W1bP, b1.reshape(1, H1))
    bnds = jnp.searchsorted(
        batch, jnp.arange(NUM_CLOUDS + 1, dtype=jnp.int32)).astype(jnp.int32)
    bnds = jnp.pad(bnds, (0, L - NUM_CLOUDS - 1))
    Gg = _sc_search_gather(pos[:, 0], pos[:, 1], pos[:, 2], batch, bnds, G)
    out = _tc_mlp(Gg, V, W2, b2.reshape(1, H2))
    return (out, pos, batch)


# speedup vs baseline: 1.1201x; 1.0025x over previous
import functools

import jax
import jax.numpy as jnp
import numpy as np
from jax import lax
from jax.experimental import pallas as pl
from jax.experimental.pallas import tpu as pltpu
from jax.experimental.pallas import tpu_sc as plsc

N = 8192
D_FEAT = 64
K = 32
H1 = 64
H2 = 128
NUM_CLOUDS = 8
L = 16
NSUB = 32
QPW = N // NSUB
QCH = 4
NCH = QPW // QCH

_R2_F = np.float32(0.2 * 0.2)
_R2_BITS = int(np.array(0.2 * 0.2, np.float32).view(np.int32))


def _sc_search_gather(posx, posy, posz, batch, bnds, G):
    mesh = plsc.VectorSubcoreMesh(core_axis_name="c", subcore_axis_name="s")

    @functools.partial(
        pl.kernel,
        mesh=mesh,
        compiler_params=pltpu.CompilerParams(needs_layout_passes=False),
        out_type=jax.ShapeDtypeStruct((N * K, 2 * D_FEAT), jnp.float32),
        scratch_types=[
            pltpu.VMEM((N + 8 * L,), jnp.float32),
            pltpu.VMEM((N + 8 * L,), jnp.float32),
            pltpu.VMEM((N + 8 * L,), jnp.float32),
            pltpu.VMEM((N + L,), jnp.int32),
            pltpu.VMEM((2 * L,), jnp.int32),
            pltpu.VMEM((N + 8 * L,), jnp.float32),
            pltpu.VMEM((N + 8 * L,), jnp.int32),
            pltpu.VMEM((QCH * K + L,), jnp.int32),
            pltpu.VMEM((QCH * K + L,), jnp.int32),
            pltpu.VMEM((QCH * K, 2 * D_FEAT), jnp.float32),
            pltpu.VMEM((QCH * K, 2 * D_FEAT), jnp.float32),
            pltpu.SemaphoreType.DMA,
            pltpu.SemaphoreType.DMA,
            pltpu.SemaphoreType.DMA,
            pltpu.SemaphoreType.DMA,
        ],
    )
    def k(posx_hbm, posy_hbm, posz_hbm, batch_hbm, bnds_hbm, g_hbm, out_hbm,
          px, py, pz, batv, bndv, cd2, cix, idxq0, idxq1, grows0, grows1,
          gs0, gs1, ws0, ws1):
        wid = lax.axis_index("s") * 2 + lax.axis_index("c")
        pltpu.sync_copy(posx_hbm, px.at[pl.ds(0, N)])
        pltpu.sync_copy(posy_hbm, py.at[pl.ds(0, N)])
        pltpu.sync_copy(posz_hbm, pz.at[pl.ds(0, N)])
        pltpu.sync_copy(batch_hbm, batv.at[pl.ds(0, N)])
        pltpu.sync_copy(bnds_hbm, bndv.at[pl.ds(0, L)])

        ii = lax.broadcasted_iota(jnp.int32, (L,), 0)
        zi = jnp.zeros((L,), jnp.int32)

        def sread(ref, idx):
            return ref[pl.ds(idx, L)][0]

        def query_body(qq, qc, idxq):
            i = wid * QPW + qc * QCH + qq
            c = sread(batv, i)
            s = sread(bndv, c)
            e = sread(bndv, c + 1)
            qx = sread(px, i)
            qy = sread(py, i)
            qz = sread(pz, i)
            ce = (s + (L - 1)) >> 4
            nbe = ((e + (L - 1)) >> 4) - ce

            def step(base, head):
                lix = base + ii
                dx = px[pl.ds(base, L)] - qx
                dy = py[pl.ds(base, L)] - qy
                dz = pz[pl.ds(base, L)] - qz
                d2 = dx * dx + dy * dy + dz * dz
                val = (d2 <= _R2_F) & (lix < e)
                if head:
                    val = val & (lix >= s) & (lix < ce * L)
                pc = plsc.all_reduce_population_count(val)[0]
                return d2, lix, val, pc

            def emit(m, vals):
                for d2, lix, val, pc in vals:
                    plsc.store_compressed(cd2.at[pl.ds(m, L)], d2, mask=val)
                    plsc.store_compressed(cix.at[pl.ds(m, L)], lix, mask=val)
                    m = m + pc
                return m

            def scan_body(blk, m):
                vals = []
                for u in range(8):
                    vals.append(step((ce + blk * 8 + u) * L, False))
                return emit(m, vals)

            m_sc = emit(0, [step((s >> 4) * L, True)])
            m_sc = lax.fori_loop(0, (nbe + 7) >> 3, scan_body, m_sc)
            nbm = (m_sc + (L - 1)) >> 4

            def do_search():
                def count_le(tb):
                    def cb(blk, acc):
                        for u in range(2):
                            base = (blk * 2 + u) * L
                            d2b = lax.bitcast_convert_type(
                                cd2[pl.ds(base, L)], jnp.int32)
                            ok = (base + ii < m_sc) & (d2b <= tb)
                            acc = acc + plsc.all_reduce_population_count(ok)
                        return acc
                    return lax.fori_loop(0, (nbm + 1) >> 1, cb, zi)

                def bs_body(t, lohi):
                    lo, hi = lohi
                    mid = lo + ((hi - lo) >> 1)
                    ge = count_le(mid) >= K
                    return (jnp.where(ge, lo, mid + 1),
                            jnp.where(ge, mid, hi))

                lo, _ = lax.fori_loop(0, 30, bs_body, (zi, zi + _R2_BITS))
                tstar = lo

                def clt(blk, acc):
                    for u in range(2):
                        base = (blk * 2 + u) * L
                        d2b = lax.bitcast_convert_type(
                            cd2[pl.ds(base, L)], jnp.int32)
                        ok = (base + ii < m_sc) & (d2b < tstar)
                        acc = acc + plsc.all_reduce_population_count(ok)
                    return acc

                n_lt = lax.fori_loop(0, (nbm + 1) >> 1, clt, zi)
                r = K - n_lt

                def count_tie_le(icv):
                    def cb(blk, acc):
                        for u in range(2):
                            base = (blk * 2 + u) * L
                            d2b = lax.bitcast_convert_type(
                                cd2[pl.ds(base, L)], jnp.int32)
                            ixv = cix[pl.ds(base, L)]
                            ok = ((base + ii < m_sc) & (d2b == tstar)
                                  & (ixv <= icv))
                            acc = acc + plsc.all_reduce_population_count(ok)
                        return acc
                    return lax.fori_loop(0, (nbm + 1) >> 1, cb, zi)

                def bs2_body(t, lohi):
                    lo, hi = lohi
                    mid = lo + ((hi - lo) >> 1)
                    ge = count_tie_le(mid) >= r
                    return (jnp.where(ge, lo, mid + 1),
                            jnp.where(ge, mid, hi))

                icut, _ = lax.fori_loop(0, 13, bs2_body, (zi, zi + (N - 1)))
                return tstar, icut

            def do_search_fast():
                bigd = jnp.int32(0x7FFFFFFF)
                dm = [jnp.where(u * L + ii < m_sc,
                                lax.bitcast_convert_type(
                                    cd2[pl.ds(u * L, L)], jnp.int32),
                                bigd) for u in range(3)]
                xv = [cix[pl.ds(u * L, L)] for u in range(3)]

                def cle(tb):
                    acc = zi
                    for u in range(3):
                        acc = acc + plsc.all_reduce_population_count(
                            dm[u] <= tb)
                    return acc

                def bs_body(t, lohi):
                    lo, hi = lohi
                    mid = lo + ((hi - lo) >> 1)
                    ge = cle(mid) >= K
                    return (jnp.where(ge, lo, mid + 1),
                            jnp.where(ge, mid, hi))

                tstar, _ = lax.fori_loop(0, 30, bs_body, (zi, zi + _R2_BITS))

                n_lt = zi
                for u in range(3):
                    n_lt = n_lt + plsc.all_reduce_population_count(
                        dm[u] < tstar)
                r = K - n_lt

                tx = [jnp.where(dm[u] == tstar, xv[u], jnp.int32(N))
                      for u in range(3)]

                def ctle(icv):
                    acc = zi
                    for u in range(3):
                        acc = acc + plsc.all_reduce_population_count(
                            tx[u] <= icv)
                    return acc

                def bs2_body(t, lohi):
                    lo, hi = lohi
                    mid = lo + ((hi - lo) >> 1)
                    ge = ctle(mid) >= r
                    return (jnp.where(ge, lo, mid + 1),
                            jnp.where(ge, mid, hi))

                icut, _ = lax.fori_loop(0, 13, bs2_body, (zi, zi + (N - 1)))
                return tstar, icut

            def no_search():
                return zi + (_R2_BITS + 1), zi + (N - 1)

            tstar, icut = lax.cond(
                m_sc > K,
                lambda: lax.cond(m_sc <= 3 * L, do_search_fast, do_search),
                no_search)

            def selpass(b, m2):
                base = b * L
                d2b = lax.bitcast_convert_type(cd2[pl.ds(base, L)], jnp.int32)
                ixv = cix[pl.ds(base, L)]
                sel = ((base + ii < m_sc)
                       & ((d2b < tstar) | ((d2b == tstar) & (ixv <= icut))))
                pc = plsc.all_reduce_population_count(sel)[0]
                plsc.store_compressed(idxq.at[pl.ds(qq * K + m2, L)],
                                      ixv, mask=sel)
                return m2 + pc

            m2 = lax.fori_loop(0, nbm, selpass, 0)

            for kk in range(K // L):
                off = qq * K + kk * L
                cur = idxq[pl.ds(off, L)]
                idxq[pl.ds(off, L)] = jnp.where(kk * L + ii < m2, cur, zi + i)
            return 0

        def compute_chunk(qc, idxq):
            lax.fori_loop(
                0, QCH,
                lambda qq, _: query_body(qq, qc, idxq), 0)

        def out_slice(qc):
            row0 = wid * (QPW * K) + qc * (QCH * K)
            return out_hbm.at[pl.ds(row0, QCH * K)]

        bufs = ((idxq0, grows0, gs0, ws0), (idxq1, grows1, gs1, ws1))

        def group_body(gidx, _):
            for b in range(2):
                qc = gidx * 2 + b
                idxq, grows, gsem, wsem = bufs[b]
                oidxq, ogrows, ogsem, owsem = bufs[1 - b]
                idxs = idxq.at[pl.ds(0, QCH * K)]
                oidxs = oidxq.at[pl.ds(0, QCH * K)]

                @pl.when(qc >= 2)
                def _():
                    pltpu.make_async_copy(grows, out_slice(qc - 2),
                                          wsem).wait()

                compute_chunk(qc, idxq)
                pltpu.async_copy(g_hbm.at[idxs], grows, gsem)

                @pl.when(qc >= 1)
                def _():
                    pltpu.make_async_copy(g_hbm.at[oidxs], ogrows,
                                          ogsem).wait()
                    pltpu.async_copy(ogrows, out_slice(qc - 1), owsem)

            return 0

        lax.fori_loop(0, NCH // 2, group_body, 0)
        pltpu.make_async_copy(g_hbm.at[idxq1.at[pl.ds(0, QCH * K)]],
                              grows1, gs1).wait()
        pltpu.async_copy(grows1, out_slice(NCH - 1), ws1)
        pltpu.make_async_copy(grows0, out_slice(NCH - 2), ws0).wait()
        pltpu.make_async_copy(grows1, out_slice(NCH - 1), ws1).wait()

    return k(posx, posy, posz, batch, bnds, G)


def _tc_precompute(x, posP, W1a, W1bP, b1):
    BR = 512

    def body(x_ref, p_ref, wa_ref, wb_ref, b_ref, g_ref, v_ref):
        pv = jnp.dot(p_ref[...], wb_ref[...],
                     preferred_element_type=jnp.float32)
        gv = (jnp.dot(x_ref[...], wa_ref[...],
                      preferred_element_type=jnp.float32)
              + pv + b_ref[...])
        g_ref[:, :H1] = gv
        g_ref[:, H1:] = jnp.zeros_like(gv)
        v_ref[...] = pv

    return pl.pallas_call(
        body,
        grid=(N // BR,),
        in_specs=[
            pl.BlockSpec((BR, D_FEAT), lambda i: (i, 0)),
            pl.BlockSpec((BR, 8), lambda i: (i, 0)),
            pl.BlockSpec((D_FEAT, H1), lambda i: (0, 0)),
            pl.BlockSpec((8, H1), lambda i: (0, 0)),
            pl.BlockSpec((1, H1), lambda i: (0, 0)),
        ],
        out_specs=[
            pl.BlockSpec((BR, 2 * H1), lambda i: (i, 0)),
            pl.BlockSpec((BR, H1), lambda i: (i, 0)),
        ],
        out_shape=[
            jax.ShapeDtypeStruct((N, 2 * H1), jnp.float32),
            jax.ShapeDtypeStruct((N, H1), jnp.float32),
        ],
    )(x, posP, W1a, W1bP, b1)


def _tc_mlp(Gg, V, W2, b2):
    BQ = 1024

    def body(gg_ref, v_ref, w2_ref, b2_ref, o_ref):
        g3 = gg_ref[:, :H1].reshape(BQ, K, H1)
        h1 = jnp.maximum(g3 - v_ref[...][:, None, :], 0.0)
        h2 = jnp.dot(h1.reshape(BQ * K, H1), w2_ref[...],
                     preferred_element_type=jnp.float32) + b2_ref[...]
        h2 = jnp.maximum(h2, 0.0)
        o_ref[...] = jnp.max(h2.reshape(BQ, K, H2), axis=1)

    return pl.pallas_call(
        body,
        grid=(N // BQ,),
        in_specs=[
            pl.BlockSpec((BQ * K, 2 * H1), lambda i: (i, 0)),
            pl.BlockSpec((BQ, H1), lambda i: (i, 0)),
            pl.BlockSpec((H1, H2), lambda i: (0, 0)),
            pl.BlockSpec((1, H2), lambda i: (0, 0)),
        ],
        out_specs=pl.BlockSpec((BQ, H2), lambda i: (i, 0)),
        out_shape=jax.ShapeDtypeStruct((N, H2), jnp.float32),
    )(Gg, V, W2, b2)


def kernel(x, pos, batch, W1, b1, W2, b2):
    W1a = W1[:D_FEAT]
    W1b = W1[D_FEAT:]
    posP = jnp.pad(pos, ((0, 0), (0, 5)))
    W1bP = jnp.pad(W1b, ((0, 5), (0, 0)))
    G, V = _tc_precompute(x, posP, W1a, W1bP, b1.reshape(1, H1))
    bnds = jnp.searchsorted(
        batch, jnp.arange(NUM_CLOUDS + 1, dtype=jnp.int32)).astype(jnp.int32)
    bnds = jnp.pad(bnds, (0, L - NUM_CLOUDS - 1))
    Gg = _sc_search_gather(pos[:, 0], pos[:, 1], pos[:, 2], batch, bnds, G)
    out = _tc_mlp(Gg, V, W2, b2.reshape(1, H2))
    return (out, pos, batch)
